# baseline (speedup 1.0000x reference)
import functools

import jax
import jax.numpy as jnp
from jax.experimental import pallas as pl
from jax.experimental.pallas import tpu as pltpu

DIM = 512
NUM_SLOTS = 64
CHUNK = 64
EPS = 1e-5
TOK = 512


def _softmax(logits):
    m = jnp.max(logits, axis=-1, keepdims=True)
    e = jnp.exp(logits - m)
    return e / jnp.sum(e, axis=-1, keepdims=True)


def _fused_kernel(x_ref, sk_ref, wk_ref, bk_ref, wq_ref, bq_ref, wv_ref,
                  bv_ref, scale_ref, g_ref, b_ref, wo_ref, bo_ref, o_ref):
    bf = jnp.bfloat16
    x = x_ref[...]
    xb = x.astype(bf)
    k = jnp.dot(xb, wk_ref[...], preferred_element_type=jnp.float32) + bk_ref[...]
    q = jnp.dot(xb, wq_ref[...], preferred_element_type=jnp.float32) + bq_ref[...]
    v = jnp.dot(xb, wv_ref[...], preferred_element_type=jnp.float32) + bv_ref[...]

    sk = sk_ref[...]
    scale = scale_ref[0, 0]
    dn = (((1,), (1,)), ((), ()))
    ww = _softmax(jax.lax.dot_general(k.astype(bf), sk, dn,
                                      preferred_element_type=jnp.float32) * scale)
    rw = _softmax(jax.lax.dot_general(q.astype(bf), sk, dn,
                                      preferred_element_type=jnp.float32) * scale)

    a = jax.lax.dot_general(rw.astype(bf), ww.astype(bf), dn,
                            preferred_element_type=jnp.float32)
    r = jax.lax.broadcasted_iota(jnp.int32, (TOK, TOK), 0)
    u = jax.lax.broadcasted_iota(jnp.int32, (TOK, TOK), 1)
    mask = (r // CHUNK == u // CHUNK) & (u <= r)
    a = jnp.where(mask, a, 0.0)
    ret = jnp.dot(a.astype(bf), v.astype(bf), preferred_element_type=jnp.float32)

    mu = jnp.mean(ret, axis=-1, keepdims=True)
    cen = ret - mu
    var = jnp.mean(cen * cen, axis=-1, keepdims=True)
    ln = cen * jax.lax.rsqrt(var + EPS) * g_ref[...] + b_ref[...]
    out = jnp.dot(ln.astype(bf), wo_ref[...],
                  preferred_element_type=jnp.float32) + bo_ref[...]
    o_ref[...] = x + out


def kernel(x, slot_keys, Wk, bk, Wq, bq, Wv, bv, scale, ln_g, ln_b, Wo, bo):
    b, s, d = x.shape
    n = b * s
    x2 = x.reshape(n, d)
    full = lambda i: (0, 0)
    wspec = pl.BlockSpec((d, d), full)
    vspec = pl.BlockSpec((1, d), full)
    out = pl.pallas_call(
        _fused_kernel,
        out_shape=jax.ShapeDtypeStruct((n, d), x.dtype),
        grid=(n // TOK,),
        in_specs=[
            pl.BlockSpec((TOK, d), lambda i: (i, 0)),
            pl.BlockSpec((NUM_SLOTS, d), full),
            wspec, vspec,
            wspec, vspec,
            wspec, vspec,
            pl.BlockSpec((1, 1), full, memory_space=pltpu.SMEM),
            vspec, vspec,
            wspec, vspec,
        ],
        out_specs=pl.BlockSpec((TOK, d), lambda i: (i, 0)),
        compiler_params=pltpu.CompilerParams(
            dimension_semantics=("parallel",),
        ),
        name="slot_memory_phasor",
    )(x2, slot_keys.astype(jnp.bfloat16), Wk.astype(jnp.bfloat16),
      bk.reshape(1, d), Wq.astype(jnp.bfloat16), bq.reshape(1, d),
      Wv.astype(jnp.bfloat16), bv.reshape(1, d), scale.reshape(1, 1),
      ln_g.reshape(1, d), ln_b.reshape(1, d), Wo.astype(jnp.bfloat16),
      bo.reshape(1, d))
    return out.reshape(b, s, d)


# speedup vs baseline: 1.3535x; 1.3535x over previous
import jax
import jax.numpy as jnp
import numpy as np
from jax.experimental import pallas as pl
from jax.experimental.pallas import tpu as pltpu

DIM = 512
NUM_SLOTS = 64
CHUNK = 64
EPS = 1e-5
TOK = 512


def _softmax(logits):
    m = jnp.max(logits, axis=-1, keepdims=True)
    e = jnp.exp(logits - m)
    return e / jnp.sum(e, axis=-1, keepdims=True)


def _fused_kernel(x_ref, mk_ref, bkl_ref, mq_ref, bql_ref, wv_ref, bv_ref,
                  mask_ref, wo_ref, bo_ref, o_ref):
    bf = jnp.bfloat16
    x = x_ref[...]
    xb = x.astype(bf)
    ww = _softmax(jnp.dot(xb, mk_ref[...],
                          preferred_element_type=jnp.float32) + bkl_ref[...])
    rw = _softmax(jnp.dot(xb, mq_ref[...],
                          preferred_element_type=jnp.float32) + bql_ref[...])
    v = jnp.dot(xb, wv_ref[...], preferred_element_type=jnp.float32) + bv_ref[...]

    dn_t = (((1,), (1,)), ((), ()))
    a = jax.lax.dot_general(rw.astype(bf), ww.astype(bf), dn_t,
                            preferred_element_type=jnp.float32) * mask_ref[...]
    ret = jnp.dot(a.astype(bf), v.astype(bf), preferred_element_type=jnp.float32)

    mu = jnp.mean(ret, axis=-1, keepdims=True)
    cen = ret - mu
    var = jnp.mean(cen * cen, axis=-1, keepdims=True)
    lnc = cen * jax.lax.rsqrt(var + EPS)
    out = jnp.dot(lnc.astype(bf), wo_ref[...],
                  preferred_element_type=jnp.float32) + bo_ref[...]
    o_ref[...] = x + out


def kernel(x, slot_keys, Wk, bk, Wq, bq, Wv, bv, scale, ln_g, ln_b, Wo, bo):
    b, s, d = x.shape
    n = b * s
    ns = slot_keys.shape[0]
    x2 = x.reshape(n, d)
    bf = jnp.bfloat16
    hi = jax.lax.Precision.HIGHEST

    sc = scale[0]
    mk = sc * jnp.dot(Wk, slot_keys.T, precision=hi)
    mq = sc * jnp.dot(Wq, slot_keys.T, precision=hi)
    bk_l = sc * jnp.dot(bk, slot_keys.T, precision=hi)
    bq_l = sc * jnp.dot(bq, slot_keys.T, precision=hi)
    wo_eff = ln_g[:, None] * Wo
    bo_eff = jnp.dot(ln_b, Wo, precision=hi) + bo

    r = np.arange(TOK)
    mask = ((r[:, None] // CHUNK == r[None, :] // CHUNK)
            & (r[None, :] <= r[:, None])).astype(np.float32)
    mask = jnp.asarray(mask)

    full = lambda i: (0, 0)
    wspec = pl.BlockSpec((d, d), full)
    lspec = pl.BlockSpec((d, ns), full)
    out = pl.pallas_call(
        _fused_kernel,
        out_shape=jax.ShapeDtypeStruct((n, d), x.dtype),
        grid=(n // TOK,),
        in_specs=[
            pl.BlockSpec((TOK, d), lambda i: (i, 0)),
            lspec, pl.BlockSpec((1, ns), full),
            lspec, pl.BlockSpec((1, ns), full),
            wspec, pl.BlockSpec((1, d), full),
            pl.BlockSpec((TOK, TOK), full),
            wspec, pl.BlockSpec((1, d), full),
        ],
        out_specs=pl.BlockSpec((TOK, d), lambda i: (i, 0)),
        compiler_params=pltpu.CompilerParams(
            dimension_semantics=("parallel",),
        ),
        name="slot_memory_phasor",
    )(x2, mk.astype(bf), bk_l.reshape(1, ns), mq.astype(bf),
      bq_l.reshape(1, ns), Wv.astype(bf), bv.reshape(1, d), mask,
      wo_eff.astype(bf), bo_eff.reshape(1, d))
    return out.reshape(b, s, d)


# speedup vs baseline: 1.4395x; 1.0635x over previous
import jax
import jax.numpy as jnp
import numpy as np
from jax.experimental import pallas as pl
from jax.experimental.pallas import tpu as pltpu

DIM = 512
NUM_SLOTS = 64
CHUNK = 64
EPS = 1e-5
TOK = 512


def _softmax0(logits):
    m = jnp.max(logits, axis=0, keepdims=True)
    e = jnp.exp(logits - m)
    return e / jnp.sum(e, axis=0, keepdims=True)


def _fused_kernel(x_ref, mk_ref, bkl_ref, mq_ref, bql_ref, wv_ref, bv_ref,
                  mask_ref, wo_ref, bo_ref, o_ref):
    bf = jnp.bfloat16
    x = x_ref[...]
    xb = x.astype(bf)
    dn_tab = (((0,), (1,)), ((), ()))
    wwt = _softmax0(jax.lax.dot_general(mk_ref[...], xb, dn_tab,
                                        preferred_element_type=jnp.float32)
                    + bkl_ref[...])
    rwt = _softmax0(jax.lax.dot_general(mq_ref[...], xb, dn_tab,
                                        preferred_element_type=jnp.float32)
                    + bql_ref[...])
    v = jnp.dot(xb, wv_ref[...], preferred_element_type=jnp.float32) + bv_ref[...]

    dn_ta = (((0,), (0,)), ((), ()))
    a = jax.lax.dot_general(rwt.astype(bf), wwt.astype(bf), dn_ta,
                            preferred_element_type=jnp.float32) * mask_ref[...]
    ret = jnp.dot(a.astype(bf), v.astype(bf), preferred_element_type=jnp.float32)

    mu = jnp.mean(ret, axis=-1, keepdims=True)
    cen = ret - mu
    var = jnp.mean(cen * cen, axis=-1, keepdims=True)
    lnc = cen * jax.lax.rsqrt(var + EPS)
    out = jnp.dot(lnc.astype(bf), wo_ref[...],
                  preferred_element_type=jnp.float32) + bo_ref[...]
    o_ref[...] = x + out


def kernel(x, slot_keys, Wk, bk, Wq, bq, Wv, bv, scale, ln_g, ln_b, Wo, bo):
    b, s, d = x.shape
    n = b * s
    ns = slot_keys.shape[0]
    x2 = x.reshape(n, d)
    bf = jnp.bfloat16
    hi = jax.lax.Precision.HIGHEST

    sc = scale[0]
    mk = sc * jnp.dot(Wk, slot_keys.T, precision=hi)
    mq = sc * jnp.dot(Wq, slot_keys.T, precision=hi)
    bk_l = sc * jnp.dot(bk, slot_keys.T, precision=hi)
    bq_l = sc * jnp.dot(bq, slot_keys.T, precision=hi)
    wo_eff = ln_g[:, None] * Wo
    bo_eff = jnp.dot(ln_b, Wo, precision=hi) + bo

    r = np.arange(TOK)
    mask = ((r[:, None] // CHUNK == r[None, :] // CHUNK)
            & (r[None, :] <= r[:, None])).astype(np.float32)
    mask = jnp.asarray(mask)

    full = lambda i: (0, 0)
    wspec = pl.BlockSpec((d, d), full)
    lspec = pl.BlockSpec((d, ns), full)
    out = pl.pallas_call(
        _fused_kernel,
        out_shape=jax.ShapeDtypeStruct((n, d), x.dtype),
        grid=(n // TOK,),
        in_specs=[
            pl.BlockSpec((TOK, d), lambda i: (i, 0)),
            lspec, pl.BlockSpec((ns, 1), full),
            lspec, pl.BlockSpec((ns, 1), full),
            wspec, pl.BlockSpec((1, d), full),
            pl.BlockSpec((TOK, TOK), full),
            wspec, pl.BlockSpec((1, d), full),
        ],
        out_specs=pl.BlockSpec((TOK, d), lambda i: (i, 0)),
        compiler_params=pltpu.CompilerParams(
            dimension_semantics=("parallel",),
        ),
        name="slot_memory_phasor",
    )(x2, mk.astype(bf), bk_l.reshape(ns, 1), mq.astype(bf),
      bq_l.reshape(ns, 1), Wv.astype(bf), bv.reshape(1, d), mask,
      wo_eff.astype(bf), bo_eff.reshape(1, d))
    return out.reshape(b, s, d)


# speedup vs baseline: 1.5201x; 1.0560x over previous
import jax
import jax.numpy as jnp
import numpy as np
from jax.experimental import pallas as pl
from jax.experimental.pallas import tpu as pltpu

DIM = 512
NUM_SLOTS = 64
CHUNK = 64
EPS = 1e-5
HALF = 512
TOK = 1024


def _softmax0(logits):
    m = jnp.max(logits, axis=0, keepdims=True)
    e = jnp.exp(logits - m)
    return e / jnp.sum(e, axis=0, keepdims=True)


def _fused_kernel(x_ref, mk_ref, bkl_ref, mq_ref, bql_ref, wv_ref, bv_ref,
                  mask_ref, wo_ref, bo_ref, o_ref):
    bf = jnp.bfloat16
    for h in range(TOK // HALF):
        rows = pl.ds(h * HALF, HALF)
        x = x_ref[rows, :]
        xb = x.astype(bf)
        dn_tab = (((0,), (1,)), ((), ()))
        wwt = _softmax0(jax.lax.dot_general(mk_ref[...], xb, dn_tab,
                                            preferred_element_type=jnp.float32)
                        + bkl_ref[...])
        rwt = _softmax0(jax.lax.dot_general(mq_ref[...], xb, dn_tab,
                                            preferred_element_type=jnp.float32)
                        + bql_ref[...])
        v = jnp.dot(xb, wv_ref[...],
                    preferred_element_type=jnp.float32) + bv_ref[...]

        dn_ta = (((0,), (0,)), ((), ()))
        a = jax.lax.dot_general(rwt.astype(bf), wwt.astype(bf), dn_ta,
                                preferred_element_type=jnp.float32) * mask_ref[...]
        ret = jnp.dot(a.astype(bf), v.astype(bf),
                      preferred_element_type=jnp.float32)

        mu = jnp.mean(ret, axis=-1, keepdims=True)
        cen = ret - mu
        var = jnp.mean(cen * cen, axis=-1, keepdims=True)
        lnc = cen * jax.lax.rsqrt(var + EPS)
        out = jnp.dot(lnc.astype(bf), wo_ref[...],
                      preferred_element_type=jnp.float32) + bo_ref[...]
        o_ref[rows, :] = x + out


def kernel(x, slot_keys, Wk, bk, Wq, bq, Wv, bv, scale, ln_g, ln_b, Wo, bo):
    b, s, d = x.shape
    n = b * s
    ns = slot_keys.shape[0]
    x2 = x.reshape(n, d)
    bf = jnp.bfloat16
    hi = jax.lax.Precision.HIGHEST

    sc = scale[0]
    mk = sc * jnp.dot(Wk, slot_keys.T, precision=hi)
    mq = sc * jnp.dot(Wq, slot_keys.T, precision=hi)
    bk_l = sc * jnp.dot(bk, slot_keys.T, precision=hi)
    bq_l = sc * jnp.dot(bq, slot_keys.T, precision=hi)
    wo_eff = ln_g[:, None] * Wo
    bo_eff = jnp.dot(ln_b, Wo, precision=hi) + bo

    r = np.arange(HALF)
    mask = ((r[:, None] // CHUNK == r[None, :] // CHUNK)
            & (r[None, :] <= r[:, None])).astype(np.float32)
    mask = jnp.asarray(mask)

    full = lambda i: (0, 0)
    wspec = pl.BlockSpec((d, d), full)
    lspec = pl.BlockSpec((d, ns), full)
    out = pl.pallas_call(
        _fused_kernel,
        out_shape=jax.ShapeDtypeStruct((n, d), x.dtype),
        grid=(n // TOK,),
        in_specs=[
            pl.BlockSpec((TOK, d), lambda i: (i, 0)),
            lspec, pl.BlockSpec((ns, 1), full),
            lspec, pl.BlockSpec((ns, 1), full),
            wspec, pl.BlockSpec((1, d), full),
            pl.BlockSpec((HALF, HALF), full),
            wspec, pl.BlockSpec((1, d), full),
        ],
        out_specs=pl.BlockSpec((TOK, d), lambda i: (i, 0)),
        compiler_params=pltpu.CompilerParams(
            dimension_semantics=("parallel",),
        ),
        name="slot_memory_phasor",
    )(x2, mk.astype(bf), bk_l.reshape(ns, 1), mq.astype(bf),
      bq_l.reshape(ns, 1), Wv.astype(bf), bv.reshape(1, d), mask,
      wo_eff.astype(bf), bo_eff.reshape(1, d))
    return out.reshape(b, s, d)


# speedup vs baseline: 1.6952x; 1.1152x over previous
import jax
import jax.numpy as jnp
import numpy as np
from jax.experimental import pallas as pl
from jax.experimental.pallas import tpu as pltpu

DIM = 512
NUM_SLOTS = 64
CHUNK = 64
EPS = 1e-5
HALF = 512
TOK = 1024


def _softmax0(logits):
    m = jnp.max(logits, axis=0, keepdims=True)
    e = jnp.exp(logits - m)
    return e / jnp.sum(e, axis=0, keepdims=True)


def _fused_kernel(x_ref, mk_ref, bkl_ref, mq_ref, bql_ref, wv_ref, bv_ref,
                  mask_ref, wo_ref, bo_ref, o_ref):
    bf = jnp.bfloat16
    x = x_ref[...]
    xb = x.astype(bf)
    dn_tab = (((0,), (1,)), ((), ()))
    wwt = _softmax0(jax.lax.dot_general(mk_ref[...], xb, dn_tab,
                                        preferred_element_type=jnp.float32)
                    + bkl_ref[...]).astype(bf)
    rwt = _softmax0(jax.lax.dot_general(mq_ref[...], xb, dn_tab,
                                        preferred_element_type=jnp.float32)
                    + bql_ref[...]).astype(bf)
    vb = (jnp.dot(xb, wv_ref[...],
                  preferred_element_type=jnp.float32) + bv_ref[...]).astype(bf)

    dn_ta = (((0,), (0,)), ((), ()))
    rets = []
    for h in range(TOK // HALF):
        lo, hi = h * HALF, (h + 1) * HALF
        a = jax.lax.dot_general(rwt[:, lo:hi], wwt[:, lo:hi], dn_ta,
                                preferred_element_type=jnp.float32) * mask_ref[...]
        rets.append(jnp.dot(a.astype(bf), vb[lo:hi, :],
                            preferred_element_type=jnp.float32))
    ret = jnp.concatenate(rets, axis=0)

    mu = jnp.mean(ret, axis=-1, keepdims=True)
    cen = ret - mu
    var = jnp.mean(cen * cen, axis=-1, keepdims=True)
    lnc = cen * jax.lax.rsqrt(var + EPS)
    out = jnp.dot(lnc.astype(bf), wo_ref[...],
                  preferred_element_type=jnp.float32) + bo_ref[...]
    o_ref[...] = x + out


def kernel(x, slot_keys, Wk, bk, Wq, bq, Wv, bv, scale, ln_g, ln_b, Wo, bo):
    b, s, d = x.shape
    n = b * s
    ns = slot_keys.shape[0]
    x2 = x.reshape(n, d)
    bf = jnp.bfloat16
    hi = jax.lax.Precision.HIGHEST

    sc = scale[0]
    mk = sc * jnp.dot(Wk, slot_keys.T, precision=hi)
    mq = sc * jnp.dot(Wq, slot_keys.T, precision=hi)
    bk_l = sc * jnp.dot(bk, slot_keys.T, precision=hi)
    bq_l = sc * jnp.dot(bq, slot_keys.T, precision=hi)
    wo_eff = ln_g[:, None] * Wo
    bo_eff = jnp.dot(ln_b, Wo, precision=hi) + bo

    r = np.arange(HALF)
    mask = ((r[:, None] // CHUNK == r[None, :] // CHUNK)
            & (r[None, :] <= r[:, None])).astype(np.float32)
    mask = jnp.asarray(mask)

    full = lambda i: (0, 0)
    wspec = pl.BlockSpec((d, d), full)
    lspec = pl.BlockSpec((d, ns), full)
    out = pl.pallas_call(
        _fused_kernel,
        out_shape=jax.ShapeDtypeStruct((n, d), x.dtype),
        grid=(n // TOK,),
        in_specs=[
            pl.BlockSpec((TOK, d), lambda i: (i, 0)),
            lspec, pl.BlockSpec((ns, 1), full),
            lspec, pl.BlockSpec((ns, 1), full),
            wspec, pl.BlockSpec((1, d), full),
            pl.BlockSpec((HALF, HALF), full),
            wspec, pl.BlockSpec((1, d), full),
        ],
        out_specs=pl.BlockSpec((TOK, d), lambda i: (i, 0)),
        compiler_params=pltpu.CompilerParams(
            dimension_semantics=("parallel",),
        ),
        name="slot_memory_phasor",
    )(x2, mk.astype(bf), bk_l.reshape(ns, 1), mq.astype(bf),
      bq_l.reshape(ns, 1), Wv.astype(bf), bv.reshape(1, d), mask,
      wo_eff.astype(bf), bo_eff.reshape(1, d))
    return out.reshape(b, s, d)


# speedup vs baseline: 1.7115x; 1.0096x over previous
import jax
import jax.numpy as jnp
import numpy as np
from jax.experimental import pallas as pl
from jax.experimental.pallas import tpu as pltpu

DIM = 512
NUM_SLOTS = 64
CHUNK = 64
EPS = 1e-5
TILE = 256
TOK = 1024


def _softmax0(logits):
    m = jnp.max(logits, axis=0, keepdims=True)
    e = jnp.exp(logits - m)
    return e / jnp.sum(e, axis=0, keepdims=True)


def _fused_kernel(x_ref, mk_ref, bkl_ref, mq_ref, bql_ref, wv_ref, bv_ref,
                  mask_ref, wo_ref, bo_ref, o_ref):
    bf = jnp.bfloat16
    x = x_ref[...]
    xb = x.astype(bf)
    dn_tab = (((0,), (1,)), ((), ()))
    wwt = _softmax0(jax.lax.dot_general(mk_ref[...], xb, dn_tab,
                                        preferred_element_type=jnp.float32)
                    + bkl_ref[...]).astype(bf)
    rwt = _softmax0(jax.lax.dot_general(mq_ref[...], xb, dn_tab,
                                        preferred_element_type=jnp.float32)
                    + bql_ref[...]).astype(bf)
    vb = (jnp.dot(xb, wv_ref[...],
                  preferred_element_type=jnp.float32) + bv_ref[...]).astype(bf)

    dn_ta = (((0,), (0,)), ((), ()))
    rets = []
    for h in range(TOK // TILE):
        lo, hi = h * TILE, (h + 1) * TILE
        a = jax.lax.dot_general(rwt[:, lo:hi], wwt[:, lo:hi], dn_ta,
                                preferred_element_type=jnp.float32) * mask_ref[...]
        rets.append(jnp.dot(a.astype(bf), vb[lo:hi, :],
                            preferred_element_type=jnp.float32))
    ret = jnp.concatenate(rets, axis=0)

    mu = jnp.mean(ret, axis=-1, keepdims=True)
    cen = ret - mu
    var = jnp.mean(cen * cen, axis=-1, keepdims=True)
    lnc = cen * jax.lax.rsqrt(var + EPS)
    out = jnp.dot(lnc.astype(bf), wo_ref[...],
                  preferred_element_type=jnp.float32) + bo_ref[...]
    o_ref[...] = x + out


def kernel(x, slot_keys, Wk, bk, Wq, bq, Wv, bv, scale, ln_g, ln_b, Wo, bo):
    b, s, d = x.shape
    n = b * s
    ns = slot_keys.shape[0]
    x2 = x.reshape(n, d)
    bf = jnp.bfloat16
    hi = jax.lax.Precision.HIGHEST

    sc = scale[0]
    mk = sc * jnp.dot(Wk, slot_keys.T, precision=hi)
    mq = sc * jnp.dot(Wq, slot_keys.T, precision=hi)
    bk_l = sc * jnp.dot(bk, slot_keys.T, precision=hi)
    bq_l = sc * jnp.dot(bq, slot_keys.T, precision=hi)
    wo_eff = ln_g[:, None] * Wo
    bo_eff = jnp.dot(ln_b, Wo, precision=hi) + bo

    r = np.arange(TILE)
    mask = ((r[:, None] // CHUNK == r[None, :] // CHUNK)
            & (r[None, :] <= r[:, None])).astype(np.float32)
    mask = jnp.asarray(mask)

    full = lambda i: (0, 0)
    wspec = pl.BlockSpec((d, d), full)
    lspec = pl.BlockSpec((d, ns), full)
    out = pl.pallas_call(
        _fused_kernel,
        out_shape=jax.ShapeDtypeStruct((n, d), x.dtype),
        grid=(n // TOK,),
        in_specs=[
            pl.BlockSpec((TOK, d), lambda i: (i, 0)),
            lspec, pl.BlockSpec((ns, 1), full),
            lspec, pl.BlockSpec((ns, 1), full),
            wspec, pl.BlockSpec((1, d), full),
            pl.BlockSpec((TILE, TILE), full),
            wspec, pl.BlockSpec((1, d), full),
        ],
        out_specs=pl.BlockSpec((TOK, d), lambda i: (i, 0)),
        compiler_params=pltpu.CompilerParams(
            dimension_semantics=("parallel",),
        ),
        name="slot_memory_phasor",
    )(x2, mk.astype(bf), bk_l.reshape(ns, 1), mq.astype(bf),
      bq_l.reshape(ns, 1), Wv.astype(bf), bv.reshape(1, d), mask,
      wo_eff.astype(bf), bo_eff.reshape(1, d))
    return out.reshape(b, s, d)


# speedup vs baseline: 1.9168x; 1.1200x over previous
import jax
import jax.numpy as jnp
import numpy as np
from jax.experimental import pallas as pl
from jax.experimental.pallas import tpu as pltpu

DIM = 512
NUM_SLOTS = 64
CHUNK = 64
EPS = 1e-5
TILE = 256
TOK = 1024


def _softmax0(logits):
    m = jnp.max(logits, axis=0, keepdims=True)
    e = jnp.exp(logits - m)
    return e / jnp.sum(e, axis=0, keepdims=True)


def _fused_kernel(x_ref, mkq_ref, bkq_ref, wv_ref, bv_ref,
                  mask_ref, wo_ref, bo_ref, o_ref):
    bf = jnp.bfloat16
    ns = NUM_SLOTS
    x = x_ref[...]
    xb = x.astype(bf)
    dn_tab = (((0,), (1,)), ((), ()))
    l2 = jax.lax.dot_general(mkq_ref[...], xb, dn_tab,
                             preferred_element_type=jnp.float32) + bkq_ref[...]
    wwt = _softmax0(l2[:ns]).astype(bf)
    rwt = _softmax0(l2[ns:]).astype(bf)
    vb = (jnp.dot(xb, wv_ref[...],
                  preferred_element_type=jnp.float32) + bv_ref[...]).astype(bf)

    dn_ta = (((0,), (0,)), ((), ()))
    rets = []
    for h in range(TOK // TILE):
        lo, hi = h * TILE, (h + 1) * TILE
        a = jax.lax.dot_general(rwt[:, lo:hi], wwt[:, lo:hi], dn_ta,
                                preferred_element_type=jnp.float32) * mask_ref[...]
        rets.append(jnp.dot(a.astype(bf), vb[lo:hi, :],
                            preferred_element_type=jnp.float32))
    ret = jnp.concatenate(rets, axis=0)

    mu = jnp.mean(ret, axis=-1, keepdims=True)
    cen = ret - mu
    var = jnp.mean(cen * cen, axis=-1, keepdims=True)
    lnc = cen * jax.lax.rsqrt(var + EPS)
    out = jnp.dot(lnc.astype(bf), wo_ref[...],
                  preferred_element_type=jnp.float32) + bo_ref[...]
    o_ref[...] = x + out


def kernel(x, slot_keys, Wk, bk, Wq, bq, Wv, bv, scale, ln_g, ln_b, Wo, bo):
    b, s, d = x.shape
    n = b * s
    ns = slot_keys.shape[0]
    x2 = x.reshape(n, d)
    bf = jnp.bfloat16
    hi = jax.lax.Precision.HIGHEST

    sc = scale[0]
    stk = jnp.concatenate([Wk, Wq, bk[None, :], bq[None, :]], axis=0)
    f = sc * jnp.dot(stk, slot_keys.T, precision=hi)
    mkq = jnp.concatenate([f[:d], f[d:2 * d]], axis=1)
    bkq = jnp.concatenate([f[2 * d], f[2 * d + 1]], axis=0)
    wo_eff = ln_g[:, None] * Wo
    bo_eff = jnp.dot(ln_b, Wo, precision=hi) + bo

    r = np.arange(TILE)
    mask = ((r[:, None] // CHUNK == r[None, :] // CHUNK)
            & (r[None, :] <= r[:, None])).astype(np.float32)
    mask = jnp.asarray(mask)

    full = lambda i: (0, 0)
    wspec = pl.BlockSpec((d, d), full)
    out = pl.pallas_call(
        _fused_kernel,
        out_shape=jax.ShapeDtypeStruct((n, d), x.dtype),
        grid=(n // TOK,),
        in_specs=[
            pl.BlockSpec((TOK, d), lambda i: (i, 0)),
            pl.BlockSpec((d, 2 * ns), full),
            pl.BlockSpec((2 * ns, 1), full),
            wspec, pl.BlockSpec((1, d), full),
            pl.BlockSpec((TILE, TILE), full),
            wspec, pl.BlockSpec((1, d), full),
        ],
        out_specs=pl.BlockSpec((TOK, d), lambda i: (i, 0)),
        compiler_params=pltpu.CompilerParams(
            dimension_semantics=("parallel",),
        ),
        name="slot_memory_phasor",
    )(x2, mkq.astype(bf), bkq.reshape(2 * ns, 1), Wv.astype(bf),
      bv.reshape(1, d), mask, wo_eff.astype(bf), bo_eff.reshape(1, d))
    return out.reshape(b, s, d)


# speedup vs baseline: 2.1157x; 1.1038x over previous
import jax
import jax.numpy as jnp
import numpy as np
from jax.experimental import pallas as pl
from jax.experimental.pallas import tpu as pltpu

DIM = 512
NUM_SLOTS = 64
CHUNK = 64
EPS = 1e-5
TILE = 256
TOK = 2048


def _softmax0(logits):
    m = jnp.max(logits, axis=0, keepdims=True)
    e = jnp.exp(logits - m)
    return e / jnp.sum(e, axis=0, keepdims=True)


def _fused_kernel(x_ref, mkq_ref, bkq_ref, wv_ref, bv_ref,
                  mask_ref, wo_ref, bo_ref, o_ref):
    bf = jnp.bfloat16
    ns = NUM_SLOTS
    x = x_ref[...]
    xb = x.astype(bf)
    dn_tab = (((0,), (1,)), ((), ()))
    l2 = jax.lax.dot_general(mkq_ref[...], xb, dn_tab,
                             preferred_element_type=jnp.float32) + bkq_ref[...]
    wwt = _softmax0(l2[:ns]).astype(bf)
    rwt = _softmax0(l2[ns:]).astype(bf)
    vb = (jnp.dot(xb, wv_ref[...],
                  preferred_element_type=jnp.float32) + bv_ref[...]).astype(bf)

    dn_ta = (((0,), (0,)), ((), ()))
    rets = []
    for h in range(TOK // TILE):
        lo, hi = h * TILE, (h + 1) * TILE
        a = jax.lax.dot_general(rwt[:, lo:hi], wwt[:, lo:hi], dn_ta,
                                preferred_element_type=jnp.float32) * mask_ref[...]
        rets.append(jnp.dot(a.astype(bf), vb[lo:hi, :],
                            preferred_element_type=jnp.float32))
    ret = jnp.concatenate(rets, axis=0)

    mu = jnp.mean(ret, axis=-1, keepdims=True)
    cen = ret - mu
    var = jnp.mean(cen * cen, axis=-1, keepdims=True)
    lnc = cen * jax.lax.rsqrt(var + EPS)
    out = jnp.dot(lnc.astype(bf), wo_ref[...],
                  preferred_element_type=jnp.float32) + bo_ref[...]
    o_ref[...] = x + out


def kernel(x, slot_keys, Wk, bk, Wq, bq, Wv, bv, scale, ln_g, ln_b, Wo, bo):
    b, s, d = x.shape
    n = b * s
    ns = slot_keys.shape[0]
    x2 = x.reshape(n, d)
    bf = jnp.bfloat16
    hi = jax.lax.Precision.HIGHEST

    sc = scale[0]
    stk = jnp.concatenate([Wk, Wq, bk[None, :], bq[None, :]], axis=0)
    f = sc * jnp.dot(stk, slot_keys.T, precision=hi)
    mkq = jnp.concatenate([f[:d], f[d:2 * d]], axis=1)
    bkq = jnp.concatenate([f[2 * d], f[2 * d + 1]], axis=0)
    wo_eff = ln_g[:, None] * Wo
    bo_eff = jnp.dot(ln_b, Wo, precision=hi) + bo

    r = np.arange(TILE)
    mask = ((r[:, None] // CHUNK == r[None, :] // CHUNK)
            & (r[None, :] <= r[:, None])).astype(np.float32)
    mask = jnp.asarray(mask)

    full = lambda i: (0, 0)
    wspec = pl.BlockSpec((d, d), full)
    out = pl.pallas_call(
        _fused_kernel,
        out_shape=jax.ShapeDtypeStruct((n, d), x.dtype),
        grid=(n // TOK,),
        in_specs=[
            pl.BlockSpec((TOK, d), lambda i: (i, 0)),
            pl.BlockSpec((d, 2 * ns), full),
            pl.BlockSpec((2 * ns, 1), full),
            wspec, pl.BlockSpec((1, d), full),
            pl.BlockSpec((TILE, TILE), full),
            wspec, pl.BlockSpec((1, d), full),
        ],
        out_specs=pl.BlockSpec((TOK, d), lambda i: (i, 0)),
        compiler_params=pltpu.CompilerParams(
            dimension_semantics=("parallel",),
            vmem_limit_bytes=52 * 1024 * 1024,
        ),
        name="slot_memory_phasor",
    )(x2, mkq.astype(bf), bkq.reshape(2 * ns, 1), Wv.astype(bf),
      bv.reshape(1, d), mask, wo_eff.astype(bf), bo_eff.reshape(1, d))
    return out.reshape(b, s, d)


# speedup vs baseline: 2.1280x; 1.0058x over previous
import jax
import jax.numpy as jnp
import numpy as np
from jax.experimental import pallas as pl
from jax.experimental.pallas import tpu as pltpu

DIM = 512
NUM_SLOTS = 64
CHUNK = 64
EPS = 1e-5
TILE = 256
TOK = 2048


def _softmax0(logits):
    m = jnp.max(logits, axis=0, keepdims=True)
    e = jnp.exp(logits - m)
    return e / jnp.sum(e, axis=0, keepdims=True)


def _fused_kernel(x_ref, mkq_ref, bkq_ref, wv_ref, bv_ref,
                  mask_ref, wo_ref, bo_ref, o_ref):
    bf = jnp.bfloat16
    ns = NUM_SLOTS
    x = x_ref[...]
    xb = x.astype(bf)
    dn_tab = (((0,), (1,)), ((), ()))
    l2 = jax.lax.dot_general(mkq_ref[...], xb, dn_tab,
                             preferred_element_type=jnp.float32) + bkq_ref[...]
    wwt = _softmax0(l2[:ns]).astype(bf)
    rwt = _softmax0(l2[ns:]).astype(bf)
    vb = (jnp.dot(xb, wv_ref[...],
                  preferred_element_type=jnp.float32) + bv_ref[...]).astype(bf)

    dn_ta = (((0,), (0,)), ((), ()))
    rets = []
    for h in range(TOK // TILE):
        lo, hi = h * TILE, (h + 1) * TILE
        a = jax.lax.dot_general(rwt[:, lo:hi], wwt[:, lo:hi], dn_ta,
                                preferred_element_type=jnp.float32)
        rets.append(jnp.dot(a.astype(bf) * mask_ref[...], vb[lo:hi, :],
                            preferred_element_type=jnp.float32))
    ret = jnp.concatenate(rets, axis=0)

    mu = jnp.mean(ret, axis=-1, keepdims=True)
    msq = jnp.mean(ret * ret, axis=-1, keepdims=True)
    rs = jax.lax.rsqrt(msq - mu * mu + EPS)
    lnc = (ret.astype(bf) - mu.astype(bf)) * rs.astype(bf)
    out = jnp.dot(lnc, wo_ref[...],
                  preferred_element_type=jnp.float32) + bo_ref[...]
    o_ref[...] = x + out


def kernel(x, slot_keys, Wk, bk, Wq, bq, Wv, bv, scale, ln_g, ln_b, Wo, bo):
    b, s, d = x.shape
    n = b * s
    ns = slot_keys.shape[0]
    x2 = x.reshape(n, d)
    bf = jnp.bfloat16
    hi = jax.lax.Precision.HIGHEST

    sc = scale[0]
    stk = jnp.concatenate([Wk, Wq, bk[None, :], bq[None, :]], axis=0)
    f = sc * jnp.dot(stk, slot_keys.T, precision=hi)
    mkq = jnp.concatenate([f[:d], f[d:2 * d]], axis=1)
    bkq = jnp.concatenate([f[2 * d], f[2 * d + 1]], axis=0)
    wo_eff = ln_g[:, None] * Wo
    bo_eff = jnp.dot(ln_b, Wo, precision=hi) + bo

    r = np.arange(TILE)
    mask = ((r[:, None] // CHUNK == r[None, :] // CHUNK)
            & (r[None, :] <= r[:, None]))
    mask = jnp.asarray(mask).astype(jnp.bfloat16)

    full = lambda i: (0, 0)
    wspec = pl.BlockSpec((d, d), full)
    out = pl.pallas_call(
        _fused_kernel,
        out_shape=jax.ShapeDtypeStruct((n, d), x.dtype),
        grid=(n // TOK,),
        in_specs=[
            pl.BlockSpec((TOK, d), lambda i: (i, 0)),
            pl.BlockSpec((d, 2 * ns), full),
            pl.BlockSpec((2 * ns, 1), full),
            wspec, pl.BlockSpec((1, d), full),
            pl.BlockSpec((TILE, TILE), full),
            wspec, pl.BlockSpec((1, d), full),
        ],
        out_specs=pl.BlockSpec((TOK, d), lambda i: (i, 0)),
        compiler_params=pltpu.CompilerParams(
            dimension_semantics=("parallel",),
            vmem_limit_bytes=52 * 1024 * 1024,
        ),
        name="slot_memory_phasor",
    )(x2, mkq.astype(bf), bkq.reshape(2 * ns, 1), Wv.astype(bf),
      bv.reshape(1, d), mask, wo_eff.astype(bf), bo_eff.reshape(1, d))
    return out.reshape(b, s, d)


# speedup vs baseline: 2.1357x; 1.0036x over previous
import jax
import jax.numpy as jnp
import numpy as np
from jax.experimental import pallas as pl
from jax.experimental.pallas import tpu as pltpu

DIM = 512
NUM_SLOTS = 64
CHUNK = 64
EPS = 1e-5
TILE = 256
TOK = 2048


def _softmax0(logits):
    m = jnp.max(logits, axis=0, keepdims=True)
    e = jnp.exp(logits - m)
    return e / jnp.sum(e, axis=0, keepdims=True)


def _fused_kernel(x_ref, mkq_ref, bkq_ref, wv_ref, bv_ref,
                  mask_ref, wo_ref, bo_ref, o_ref):
    bf = jnp.bfloat16
    ns = NUM_SLOTS
    x = x_ref[...]
    xb = x.astype(bf)
    dn_tab = (((0,), (1,)), ((), ()))
    l2 = jax.lax.dot_general(mkq_ref[...], xb, dn_tab,
                             preferred_element_type=jnp.float32) + bkq_ref[...]
    wwt = _softmax0(l2[:ns]).astype(bf)
    rwt = _softmax0(l2[ns:]).astype(bf)
    vb = (jnp.dot(xb, wv_ref[...],
                  preferred_element_type=jnp.float32) + bv_ref[...]).astype(bf)

    dn_ta = (((0,), (0,)), ((), ()))
    lncs = []
    for h in range(TOK // TILE):
        lo, hi = h * TILE, (h + 1) * TILE
        a = jax.lax.dot_general(rwt[:, lo:hi], wwt[:, lo:hi], dn_ta,
                                preferred_element_type=jnp.float32)
        ret = jnp.dot(a.astype(bf) * mask_ref[...], vb[lo:hi, :],
                      preferred_element_type=jnp.float32)
        mu = jnp.mean(ret, axis=-1, keepdims=True)
        msq = jnp.mean(ret * ret, axis=-1, keepdims=True)
        rs = jax.lax.rsqrt(msq - mu * mu + EPS)
        lncs.append((ret.astype(bf) - mu.astype(bf)) * rs.astype(bf))
    lnc = jnp.concatenate(lncs, axis=0)
    out = jnp.dot(lnc, wo_ref[...],
                  preferred_element_type=jnp.float32) + bo_ref[...]
    o_ref[...] = x + out


def kernel(x, slot_keys, Wk, bk, Wq, bq, Wv, bv, scale, ln_g, ln_b, Wo, bo):
    b, s, d = x.shape
    n = b * s
    ns = slot_keys.shape[0]
    x2 = x.reshape(n, d)
    bf = jnp.bfloat16
    hi = jax.lax.Precision.HIGHEST

    sc = scale[0]
    stk = jnp.concatenate([Wk, Wq, bk[None, :], bq[None, :]], axis=0)
    f = sc * jnp.dot(stk, slot_keys.T, precision=hi)
    mkq = jnp.concatenate([f[:d], f[d:2 * d]], axis=1)
    bkq = jnp.concatenate([f[2 * d], f[2 * d + 1]], axis=0)
    wo_eff = ln_g[:, None] * Wo
    bo_eff = jnp.dot(ln_b, Wo, precision=hi) + bo

    r = np.arange(TILE)
    mask = ((r[:, None] // CHUNK == r[None, :] // CHUNK)
            & (r[None, :] <= r[:, None]))
    mask = jnp.asarray(mask).astype(jnp.bfloat16)

    full = lambda i: (0, 0)
    wspec = pl.BlockSpec((d, d), full)
    out = pl.pallas_call(
        _fused_kernel,
        out_shape=jax.ShapeDtypeStruct((n, d), x.dtype),
        grid=(n // TOK,),
        in_specs=[
            pl.BlockSpec((TOK, d), lambda i: (i, 0)),
            pl.BlockSpec((d, 2 * ns), full),
            pl.BlockSpec((2 * ns, 1), full),
            wspec, pl.BlockSpec((1, d), full),
            pl.BlockSpec((TILE, TILE), full),
            wspec, pl.BlockSpec((1, d), full),
        ],
        out_specs=pl.BlockSpec((TOK, d), lambda i: (i, 0)),
        compiler_params=pltpu.CompilerParams(
            dimension_semantics=("parallel",),
            vmem_limit_bytes=52 * 1024 * 1024,
        ),
        name="slot_memory_phasor",
    )(x2, mkq.astype(bf), bkq.reshape(2 * ns, 1), Wv.astype(bf),
      bv.reshape(1, d), mask, wo_eff.astype(bf), bo_eff.reshape(1, d))
    return out.reshape(b, s, d)
